# baseline (speedup 1.0000x reference)
import functools

import jax
import jax.numpy as jnp
from jax import lax
from jax.experimental import pallas as pl
from jax.experimental.pallas import tpu as pltpu
from jax.experimental.pallas import tpu_sc as plsc

NUM_CORES = 2
NUM_SUBCORES = 16
LANES = 16
NUM_WORKERS = NUM_CORES * NUM_SUBCORES
BATCH = 16384
DIM = 32
LINE = 128
BPW = BATCH // NUM_WORKERS
CHUNK = 256
N_CHUNKS = BPW // CHUNK


def _mf_body(uidx_hbm, iidx_hbm, u_hbm, i_hbm, ubias_hbm, ibias_hbm, gbias_hbm,
             out_hbm,
             uidx_v, iidx_v, ulines_v, ilines_v, ubuf_v, ibuf_v,
             ub_v, ib_v, out_v, gb_v, sem, bsem):
    wid = lax.axis_index("s") * NUM_CORES + lax.axis_index("c")
    base = wid * BPW

    pltpu.sync_copy(uidx_hbm.at[pl.ds(base, BPW)], uidx_v)
    pltpu.sync_copy(iidx_hbm.at[pl.ds(base, BPW)], iidx_v)
    pltpu.sync_copy(gbias_hbm, gb_v)

    b0 = pltpu.async_copy(ubias_hbm.at[uidx_v], ub_v, bsem)
    b1 = pltpu.async_copy(ibias_hbm.at[iidx_v], ib_v, bsem)

    def mk_lines(k, carry):
        ulines_v[pl.ds(k * LANES, LANES)] = jnp.right_shift(
            uidx_v[pl.ds(k * LANES, LANES)], 2)
        ilines_v[pl.ds(k * LANES, LANES)] = jnp.right_shift(
            iidx_v[pl.ds(k * LANES, LANES)], 2)
        return carry

    lax.fori_loop(0, BPW // LANES, mk_lines, 0)

    b0.wait()
    b1.wait()
    g = gb_v[...]

    for c in range(N_CHUNKS):
        cbase = c * CHUNK
        c0 = pltpu.async_copy(u_hbm.at[ulines_v.at[pl.ds(cbase, CHUNK)]],
                              ubuf_v, sem)
        c1 = pltpu.async_copy(i_hbm.at[ilines_v.at[pl.ds(cbase, CHUNK)]],
                              ibuf_v, sem)
        c0.wait()
        c1.wait()

        def blk(b, carry):
            off = cbase + b * LANES
            j = lax.iota(jnp.int32, LANES) + b * LANES
            usub = jnp.left_shift(
                jnp.bitwise_and(uidx_v[pl.ds(off, LANES)], 3), 5)
            isub = jnp.left_shift(
                jnp.bitwise_and(iidx_v[pl.ds(off, LANES)], 3), 5)
            acc = ub_v[pl.ds(off, LANES)] + ib_v[pl.ds(off, LANES)] + g
            for d in range(DIM):
                acc = acc + (plsc.load_gather(ubuf_v, [j, usub + d]) *
                             plsc.load_gather(ibuf_v, [j, isub + d]))
            out_v[pl.ds(off, LANES)] = acc
            return carry

        lax.fori_loop(0, CHUNK // LANES, blk, 0)

    pltpu.sync_copy(out_v, out_hbm.at[pl.ds(base, BPW)])


@functools.partial(jax.jit, donate_argnums=())
def kernel(u_idx, i_idx, U, I, user_bias, item_bias, global_bias):
    mesh = plsc.VectorSubcoreMesh(core_axis_name="c", subcore_axis_name="s",
                                  num_cores=NUM_CORES,
                                  num_subcores=NUM_SUBCORES)
    run = pl.kernel(
        _mf_body,
        out_type=jax.ShapeDtypeStruct((BATCH,), jnp.float32),
        mesh=mesh,
        scratch_types=[
            pltpu.VMEM((BPW,), jnp.int32),
            pltpu.VMEM((BPW,), jnp.int32),
            pltpu.VMEM((BPW,), jnp.int32),
            pltpu.VMEM((BPW,), jnp.int32),
            pltpu.VMEM((CHUNK, LINE), jnp.float32),
            pltpu.VMEM((CHUNK, LINE), jnp.float32),
            pltpu.VMEM((BPW,), jnp.float32),
            pltpu.VMEM((BPW,), jnp.float32),
            pltpu.VMEM((BPW,), jnp.float32),
            pltpu.VMEM((LANES,), jnp.float32),
            pltpu.SemaphoreType.DMA,
            pltpu.SemaphoreType.DMA,
        ],
        compiler_params=pltpu.CompilerParams(needs_layout_passes=False),
    )
    u_lines = jnp.reshape(U, (U.shape[0] * DIM // LINE, LINE))
    i_lines = jnp.reshape(I, (I.shape[0] * DIM // LINE, LINE))
    gb = jnp.full((LANES,), global_bias, dtype=jnp.float32)
    return run(u_idx.astype(jnp.int32), i_idx.astype(jnp.int32),
               u_lines, i_lines, user_bias, item_bias, gb)


# speedup vs baseline: 3.3268x; 3.3268x over previous
import functools

import jax
import jax.numpy as jnp
from jax import lax
from jax.experimental import pallas as pl
from jax.experimental.pallas import tpu as pltpu
from jax.experimental.pallas import tpu_sc as plsc

NUM_CORES = 2
NUM_SUBCORES = 16
LANES = 16
NUM_WORKERS = NUM_CORES * NUM_SUBCORES
BATCH = 16384
DIM = 32
BPW = BATCH // NUM_WORKERS
CHUNK = 8
N_CHUNKS = BPW // CHUNK


def _mf_body(uidx_hbm, iidx_hbm, wu_hbm, wi_hbm, ubias_hbm, ibias_hbm,
             gbias_hbm, out_hbm,
             uidx_v, iidx_v, ublk_v, iblk_v, upack_v, ipack_v,
             ub_v, ib_v, out_v, gb_v, sem, bsem):
    wid = lax.axis_index("s") * NUM_CORES + lax.axis_index("c")
    base = wid * BPW

    pltpu.sync_copy(uidx_hbm.at[pl.ds(base, BPW)], uidx_v)
    pltpu.sync_copy(iidx_hbm.at[pl.ds(base, BPW)], iidx_v)
    pltpu.sync_copy(gbias_hbm, gb_v)

    b0 = pltpu.async_copy(ubias_hbm.at[uidx_v], ub_v, bsem)
    b1 = pltpu.async_copy(ibias_hbm.at[iidx_v], ib_v, bsem)

    cvec0 = lax.iota(jnp.int32, LANES)
    cvec1 = cvec0 + LANES

    def chunk_body(cc, carry):
        uvec = uidx_v[pl.ds(cc * LANES, LANES)]
        ivec = iidx_v[pl.ds(cc * LANES, LANES)]
        tu_vec = jnp.right_shift(uvec, 7)
        ti_vec = jnp.right_shift(ivec, 7)
        lu_vec = jnp.bitwise_and(uvec, 127)
        li_vec = jnp.bitwise_and(ivec, 127)
        for half in range(LANES // CHUNK):
            for k in range(CHUNK):
                e = half * CHUNK + k
                tu = tu_vec[e] * 128
                ti = ti_vec[e] * 128
                pltpu.async_copy(
                    wu_hbm.at[:, pl.ds(pl.multiple_of(tu, 128), 128)],
                    ublk_v.at[pl.ds(k * DIM, DIM)], sem)
                pltpu.async_copy(
                    wi_hbm.at[:, pl.ds(pl.multiple_of(ti, 128), 128)],
                    iblk_v.at[pl.ds(k * DIM, DIM)], sem)
            for k in range(CHUNK):
                pltpu.make_async_copy(wu_hbm.at[:, pl.ds(0, 128)],
                                      ublk_v.at[pl.ds(k * DIM, DIM)], sem).wait()
                pltpu.make_async_copy(wi_hbm.at[:, pl.ds(0, 128)],
                                      iblk_v.at[pl.ds(k * DIM, DIM)], sem).wait()
            for k in range(CHUNK):
                e = half * CHUNK + k
                b = cc * LANES + e
                luv = jnp.zeros((LANES,), jnp.int32) + lu_vec[e]
                liv = jnp.zeros((LANES,), jnp.int32) + li_vec[e]
                r0 = k * DIM + cvec0
                r1 = k * DIM + cvec1
                prow = jnp.right_shift(b * DIM, 7)
                pcol = jnp.bitwise_and(b * DIM, 127)
                upack_v[prow, pl.ds(pcol, LANES)] = plsc.load_gather(
                    ublk_v, [r0, luv])
                upack_v[prow, pl.ds(pcol + LANES, LANES)] = plsc.load_gather(
                    ublk_v, [r1, luv])
                ipack_v[prow, pl.ds(pcol, LANES)] = plsc.load_gather(
                    iblk_v, [r0, liv])
                ipack_v[prow, pl.ds(pcol + LANES, LANES)] = plsc.load_gather(
                    iblk_v, [r1, liv])
        return carry

    lax.fori_loop(0, BPW // LANES, chunk_body, 0)

    b0.wait()
    b1.wait()
    g = gb_v[...]

    def blk(bb, carry):
        off = bb * LANES
        rbase = (off + cvec0) * DIM
        acc = ub_v[pl.ds(off, LANES)] + ib_v[pl.ds(off, LANES)] + g
        for d in range(DIM):
            n = rbase + d
            nr = jnp.right_shift(n, 7)
            nc = jnp.bitwise_and(n, 127)
            acc = acc + (plsc.load_gather(upack_v, [nr, nc]) *
                         plsc.load_gather(ipack_v, [nr, nc]))
        out_v[pl.ds(off, LANES)] = acc
        return carry

    lax.fori_loop(0, BPW // LANES, blk, 0)

    pltpu.sync_copy(out_v, out_hbm.at[pl.ds(base, BPW)])


@functools.partial(jax.jit, donate_argnums=())
def kernel(u_idx, i_idx, U, I, user_bias, item_bias, global_bias):
    mesh = plsc.VectorSubcoreMesh(core_axis_name="c", subcore_axis_name="s",
                                  num_cores=NUM_CORES,
                                  num_subcores=NUM_SUBCORES)
    run = pl.kernel(
        _mf_body,
        out_type=jax.ShapeDtypeStruct((BATCH,), jnp.float32),
        mesh=mesh,
        scratch_types=[
            pltpu.VMEM((BPW,), jnp.int32),
            pltpu.VMEM((BPW,), jnp.int32),
            pltpu.VMEM((CHUNK * DIM, 128), jnp.float32),
            pltpu.VMEM((CHUNK * DIM, 128), jnp.float32),
            pltpu.VMEM((BPW * DIM // 128, 128), jnp.float32),
            pltpu.VMEM((BPW * DIM // 128, 128), jnp.float32),
            pltpu.VMEM((BPW,), jnp.float32),
            pltpu.VMEM((BPW,), jnp.float32),
            pltpu.VMEM((BPW,), jnp.float32),
            pltpu.VMEM((LANES,), jnp.float32),
            pltpu.SemaphoreType.DMA,
            pltpu.SemaphoreType.DMA,
        ],
        compiler_params=pltpu.CompilerParams(needs_layout_passes=False),
    )
    gb = jnp.full((LANES,), global_bias, dtype=jnp.float32)
    return run(u_idx.astype(jnp.int32), i_idx.astype(jnp.int32),
               U.T, I.T, user_bias, item_bias, gb)


# speedup vs baseline: 3.7125x; 1.1159x over previous
import functools

import jax
import jax.numpy as jnp
from jax import lax
from jax.experimental import pallas as pl
from jax.experimental.pallas import tpu as pltpu
from jax.experimental.pallas import tpu_sc as plsc

NUM_CORES = 2
NUM_SUBCORES = 16
LANES = 16
NUM_WORKERS = NUM_CORES * NUM_SUBCORES
BATCH = 16384
DIM = 32
BPW = BATCH // NUM_WORKERS
CHUNK = 8
N_CHUNKS = BPW // CHUNK


def _mf_body(uidx_hbm, iidx_hbm, wu_hbm, wi_hbm, ubias_hbm, ibias_hbm,
             gbias_hbm, out_hbm,
             uidx_v, iidx_v, ublk_v, iblk_v, upack_v, ipack_v,
             ub_v, ib_v, out_v, gb_v, usems, isems, bsem):
    wid = lax.axis_index("s") * NUM_CORES + lax.axis_index("c")
    base = wid * BPW

    pltpu.sync_copy(uidx_hbm.at[pl.ds(base, BPW)], uidx_v)
    pltpu.sync_copy(iidx_hbm.at[pl.ds(base, BPW)], iidx_v)
    pltpu.sync_copy(gbias_hbm, gb_v)

    b0 = pltpu.async_copy(ubias_hbm.at[uidx_v], ub_v, bsem)
    b1 = pltpu.async_copy(ibias_hbm.at[iidx_v], ib_v, bsem)

    cvec0 = lax.iota(jnp.int32, LANES)
    cvec1 = cvec0 + LANES

    def fire(k, tvec, e, tbl_hbm, blk_v, sems):
        t = tvec[e] * 128
        pltpu.async_copy(
            tbl_hbm.at[:, pl.ds(pl.multiple_of(t, 128), 128)],
            blk_v.at[pl.ds(k * DIM, DIM)], sems[k])

    uvec0 = uidx_v[pl.ds(0, LANES)]
    ivec0 = iidx_v[pl.ds(0, LANES)]
    tu0 = jnp.right_shift(uvec0, 7)
    ti0 = jnp.right_shift(ivec0, 7)
    for k in range(CHUNK):
        fire(k, tu0, k, wu_hbm, ublk_v, usems)
        fire(k, ti0, k, wi_hbm, iblk_v, isems)

    def chunk_body(cc, carry):
        uvec = uidx_v[pl.ds(cc * LANES, LANES)]
        ivec = iidx_v[pl.ds(cc * LANES, LANES)]
        nxt = jnp.minimum(cc + 1, BPW // LANES - 1) * LANES
        uvecn = uidx_v[pl.ds(nxt, LANES)]
        ivecn = iidx_v[pl.ds(nxt, LANES)]
        tu_vec = jnp.right_shift(uvec, 7)
        ti_vec = jnp.right_shift(ivec, 7)
        tun_vec = jnp.right_shift(uvecn, 7)
        tin_vec = jnp.right_shift(ivecn, 7)
        lu_vec = jnp.bitwise_and(uvec, 127)
        li_vec = jnp.bitwise_and(ivec, 127)
        not_last = cc < BPW // LANES - 1
        for e in range(LANES):
            k = e % CHUNK
            b = cc * LANES + e
            pltpu.make_async_copy(wu_hbm.at[:, pl.ds(0, 128)],
                                  ublk_v.at[pl.ds(k * DIM, DIM)],
                                  usems[k]).wait()
            pltpu.make_async_copy(wi_hbm.at[:, pl.ds(0, 128)],
                                  iblk_v.at[pl.ds(k * DIM, DIM)],
                                  isems[k]).wait()
            luv = jnp.zeros((LANES,), jnp.int32) + lu_vec[e]
            liv = jnp.zeros((LANES,), jnp.int32) + li_vec[e]
            r0 = k * DIM + cvec0
            r1 = k * DIM + cvec1
            prow = jnp.right_shift(b * DIM, 7)
            pcol = jnp.bitwise_and(b * DIM, 127)
            upack_v[prow, pl.ds(pcol, LANES)] = plsc.load_gather(
                ublk_v, [r0, luv])
            upack_v[prow, pl.ds(pcol + LANES, LANES)] = plsc.load_gather(
                ublk_v, [r1, luv])
            ipack_v[prow, pl.ds(pcol, LANES)] = plsc.load_gather(
                iblk_v, [r0, liv])
            ipack_v[prow, pl.ds(pcol + LANES, LANES)] = plsc.load_gather(
                iblk_v, [r1, liv])
            if e + CHUNK < LANES:
                fire(k, tu_vec, e + CHUNK, wu_hbm, ublk_v, usems)
                fire(k, ti_vec, e + CHUNK, wi_hbm, iblk_v, isems)
            else:

                @pl.when(not_last)
                def _():
                    fire(k, tun_vec, e + CHUNK - LANES, wu_hbm, ublk_v, usems)
                    fire(k, tin_vec, e + CHUNK - LANES, wi_hbm, iblk_v, isems)
        return carry

    lax.fori_loop(0, BPW // LANES, chunk_body, 0)

    b0.wait()
    b1.wait()
    g = gb_v[...]

    def blk(bb, carry):
        off = bb * LANES
        rbase = (off + cvec0) * DIM
        acc = ub_v[pl.ds(off, LANES)] + ib_v[pl.ds(off, LANES)] + g
        for d in range(DIM):
            n = rbase + d
            nr = jnp.right_shift(n, 7)
            nc = jnp.bitwise_and(n, 127)
            acc = acc + (plsc.load_gather(upack_v, [nr, nc]) *
                         plsc.load_gather(ipack_v, [nr, nc]))
        out_v[pl.ds(off, LANES)] = acc
        return carry

    lax.fori_loop(0, BPW // LANES, blk, 0)

    pltpu.sync_copy(out_v, out_hbm.at[pl.ds(base, BPW)])


@functools.partial(jax.jit, donate_argnums=())
def kernel(u_idx, i_idx, U, I, user_bias, item_bias, global_bias):
    mesh = plsc.VectorSubcoreMesh(core_axis_name="c", subcore_axis_name="s",
                                  num_cores=NUM_CORES,
                                  num_subcores=NUM_SUBCORES)
    run = pl.kernel(
        _mf_body,
        out_type=jax.ShapeDtypeStruct((BATCH,), jnp.float32),
        mesh=mesh,
        scratch_types=[
            pltpu.VMEM((BPW,), jnp.int32),
            pltpu.VMEM((BPW,), jnp.int32),
            pltpu.VMEM((CHUNK * DIM, 128), jnp.float32),
            pltpu.VMEM((CHUNK * DIM, 128), jnp.float32),
            pltpu.VMEM((BPW * DIM // 128, 128), jnp.float32),
            pltpu.VMEM((BPW * DIM // 128, 128), jnp.float32),
            pltpu.VMEM((BPW,), jnp.float32),
            pltpu.VMEM((BPW,), jnp.float32),
            pltpu.VMEM((BPW,), jnp.float32),
            pltpu.VMEM((LANES,), jnp.float32),
            [pltpu.SemaphoreType.DMA] * CHUNK,
            [pltpu.SemaphoreType.DMA] * CHUNK,
            pltpu.SemaphoreType.DMA,
        ],
        compiler_params=pltpu.CompilerParams(needs_layout_passes=False),
    )
    gb = jnp.full((LANES,), global_bias, dtype=jnp.float32)
    return run(u_idx.astype(jnp.int32), i_idx.astype(jnp.int32),
               U.T, I.T, user_bias, item_bias, gb)
